# Initial kernel scaffold; baseline (speedup 1.0000x reference)
import jax
import jax.numpy as jnp
from jax.experimental import pallas as pl


def kernel(x, edge_index, batch, Wl1, bl1, Wr1, br1, att1, bias1, Wl2, bl2, Wr2, br2, att2, bias2, W3, b3, W4, b4):
    raise NotImplementedError("write your pallas kernel here")


# speedup vs baseline: 10.0492x; 10.0492x over previous
import jax
import jax.numpy as jnp
from jax import lax
from jax.experimental import pallas as pl
from jax.experimental.pallas import tpu as pltpu
import jax.experimental.pallas.tpu_sc as plsc

N = 10000
E = 320000
IN = 128
NC = 2
NS = 16
B = 80
IBLK = 5
NBLK1 = E // (NS * IBLK * B)
NBLK2 = E // (NC * NS * IBLK * B)
NPAD = 10112
NPT = NPAD // NS
DND = 640
DPT = DND // NS


def _lanesum(v):
    idx = lax.broadcasted_iota(jnp.int32, (16,), 0)
    for k in (1, 2, 4, 8):
        v = v + v.at[idx ^ k].get(mode="promise_in_bounds", unique_indices=True)
    return v


def _proj_body(x_ref, w_ref, b_ref, o_ref):
    o_ref[...] = (
        jnp.dot(x_ref[...], w_ref[...], preferred_element_type=jnp.float32)
        + b_ref[...]
    )


def _stage_a(x, w, b):
    return pl.pallas_call(
        _proj_body,
        grid=(10,),
        in_specs=[
            pl.BlockSpec((1000, IN), lambda i: (i, 0)),
            pl.BlockSpec((IN, 512), lambda i: (0, 0)),
            pl.BlockSpec((1, 512), lambda i: (0, 0)),
        ],
        out_specs=pl.BlockSpec((1000, 512), lambda i: (i, 0)),
        out_shape=jax.ShapeDtypeStruct((N, 512), jnp.float32),
    )(x, w, b)


def _gat1_body(src_hbm, dst_hbm, xl_hbm, xr_hbm, att_hbm, zero_hbm,
               num_out, den_out,
               srcv, dstv, dgv, dhv, xlg, xrg, msg, attv,
               num_sp, den_sp, sem1, sem2):
    c = lax.axis_index("c")
    s = lax.axis_index("s")
    lane = lax.broadcasted_iota(jnp.int32, (16,), 0)
    pltpu.sync_copy(zero_hbm.at[pl.ds(s * NPT, NPT)],
                    num_sp.at[pl.ds(s * NPT, NPT)])
    pltpu.sync_copy(zero_hbm.at[pl.ds(s * DPT, DPT)],
                    den_sp.at[pl.ds(s * DPT, DPT)])
    pltpu.sync_copy(att_hbm.at[pl.ds(c * 8, 8)], attv)
    plsc.subcore_barrier()
    off = c * N

    def _blk(t, _):
        pltpu.sync_copy(src_hbm.at[s, t], srcv)
        pltpu.sync_copy(dst_hbm.at[s, t], dstv)

        def _adj(i, _a):
            for k in range(B // 16):
                d = dstv[i, pl.ds(k * 16, 16)]
                dgv[i, pl.ds(k * 16, 16)] = d + off
                dhv[i, pl.ds(k * 16, 16)] = d >> 4
                srcv[i, pl.ds(k * 16, 16)] = srcv[i, pl.ds(k * 16, 16)] + off
            return _a

        lax.fori_loop(0, IBLK, _adj, None)

        def _chunk(j, _c):
            cp1 = pltpu.async_copy(xl_hbm.at[srcv.at[j]], xlg, sem1)
            cp2 = pltpu.async_copy(xr_hbm.at[dgv.at[j]], xrg, sem2)
            cp1.wait()
            cp2.wait()

            def _edge(b, _2):
                exs = jnp.zeros((16,), jnp.float32)
                dsp = plsc.load_gather(
                    dstv,
                    [jnp.broadcast_to(j, (16,)), jnp.broadcast_to(b, (16,))])
                par8 = (dsp & 1) * 8
                grp = (dsp >> 1) & 7
                for h in range(8):
                    xlv = xlg[b, pl.ds(h * 16, 16)]
                    xrv = xrg[b, pl.ds(h * 16, 16)]
                    sv = xlv + xrv
                    ev = jnp.where(sv >= 0.0, sv, 0.2 * sv)
                    exv = jnp.exp(_lanesum(ev * attv[h, :]))
                    msg[b, pl.ds(h * 16, 16)] = exv * xlv
                    exs = jnp.where(lane == par8 + h, exv, exs)
                for g in range(8):
                    xrg[b, pl.ds(g * 16, 16)] = jnp.where(grp == g, exs, 0.0)
                return _2

            lax.fori_loop(0, B, _edge, None)
            pltpu.sync_copy(msg, num_sp.at[dstv.at[j]], add=True)
            pltpu.sync_copy(xrg, den_sp.at[dhv.at[j]], add=True)
            return _c

        lax.fori_loop(0, IBLK, _chunk, None)
        return _

    lax.fori_loop(0, NBLK1, _blk, None)
    plsc.subcore_barrier()
    pltpu.sync_copy(num_sp.at[pl.ds(s * NPT, NPT)],
                    num_out.at[c, pl.ds(s * NPT, NPT)])
    pltpu.sync_copy(den_sp.at[pl.ds(s * DPT, DPT)],
                    den_out.at[c, pl.ds(s * DPT, DPT)])


_stage_b = pl.kernel(
    _gat1_body,
    out_type=[
        jax.ShapeDtypeStruct((NC, NPAD, 128), jnp.float32),
        jax.ShapeDtypeStruct((NC, DND, 128), jnp.float32),
    ],
    mesh=plsc.VectorSubcoreMesh(core_axis_name="c", subcore_axis_name="s"),
    compiler_params=pltpu.CompilerParams(needs_layout_passes=False),
    scratch_types=[
        pltpu.VMEM((IBLK, B), jnp.int32),
        pltpu.VMEM((IBLK, B), jnp.int32),
        pltpu.VMEM((IBLK, B), jnp.int32),
        pltpu.VMEM((IBLK, B), jnp.int32),
        pltpu.VMEM((B, 128), jnp.float32),
        pltpu.VMEM((B, 128), jnp.float32),
        pltpu.VMEM((B, 128), jnp.float32),
        pltpu.VMEM((8, 16), jnp.float32),
        pltpu.VMEM_SHARED((NPAD, 128), jnp.float32),
        pltpu.VMEM_SHARED((DND, 128), jnp.float32),
        pltpu.SemaphoreType.DMA,
        pltpu.SemaphoreType.DMA,
    ],
)


def _mid_body(a0_ref, a1_ref, d0_ref, d1_ref, b1_ref, wl_ref, bl_ref,
              wr_ref, br_ref, t_ref):
    num = jnp.concatenate([a0_ref[...], a1_ref[...]], axis=1)
    den16 = jnp.concatenate([d0_ref[...], d1_ref[...]], axis=1)
    rows = num.shape[0]
    den = jnp.reshape(
        jnp.broadcast_to(den16[:, :, None], (rows, 16, 16)), (rows, 256)
    )
    h = num / jnp.where(den == 0.0, 1.0, den) + b1_ref[...]
    h = jnp.where(h > 0.0, h, jnp.exp(h) - 1.0)
    xl2 = jnp.dot(h, wl_ref[...], preferred_element_type=jnp.float32) + bl_ref[...]
    xr2 = jnp.dot(h, wr_ref[...], preferred_element_type=jnp.float32) + br_ref[...]
    t_ref[...] = jnp.concatenate(
        [xl2,
         jnp.ones((rows, 1), jnp.float32), jnp.zeros((rows, 15), jnp.float32),
         xr2,
         jnp.zeros((rows, 80), jnp.float32)],
        axis=1,
    )


def _stage_c(a0, a1, d0, d1, b1, wl, bl, wr, br):
    return pl.pallas_call(
        _mid_body,
        grid=(10,),
        in_specs=[
            pl.BlockSpec((1000, 128), lambda i: (i, 0)),
            pl.BlockSpec((1000, 128), lambda i: (i, 0)),
            pl.BlockSpec((1000, 8), lambda i: (i, 0)),
            pl.BlockSpec((1000, 8), lambda i: (i, 0)),
            pl.BlockSpec((1, 256), lambda i: (0, 0)),
            pl.BlockSpec((256, 16), lambda i: (0, 0)),
            pl.BlockSpec((1, 16), lambda i: (0, 0)),
            pl.BlockSpec((256, 16), lambda i: (0, 0)),
            pl.BlockSpec((1, 16), lambda i: (0, 0)),
        ],
        out_specs=pl.BlockSpec((1000, 128), lambda i: (i, 0)),
        out_shape=jax.ShapeDtypeStruct((N, 128), jnp.float32),
    )(a0, a1, d0, d1, b1, wl, bl, wr, br)


def _gat2_body(src_hbm, dst_hbm, t_hbm, att_hbm, zero_hbm, out_hbm,
               srcv, dstv, xlg, xrg, msg, attv, num_sp, sem1, sem2):
    c = lax.axis_index("c")
    s = lax.axis_index("s")
    pltpu.sync_copy(zero_hbm.at[pl.ds(s * NPT, NPT)],
                    num_sp.at[pl.ds(s * NPT, NPT)])
    pltpu.sync_copy(att_hbm, attv)

    def _mzero(b, _):
        for k in range(2, 8):
            msg[b, pl.ds(k * 16, 16)] = jnp.zeros((16,), jnp.float32)
        return _

    lax.fori_loop(0, B, _mzero, None)
    plsc.subcore_barrier()
    w = c * NS + s

    def _blk(t, _):
        pltpu.sync_copy(src_hbm.at[w, t], srcv)
        pltpu.sync_copy(dst_hbm.at[w, t], dstv)

        def _chunk(j, _c):
            cp1 = pltpu.async_copy(t_hbm.at[srcv.at[j]], xlg, sem1)
            cp2 = pltpu.async_copy(t_hbm.at[dstv.at[j]], xrg, sem2)
            cp1.wait()
            cp2.wait()

            def _edge(b, _2):
                xlv = xlg[b, pl.ds(0, 16)]
                auxv = xlg[b, pl.ds(16, 16)]
                xrv = xrg[b, pl.ds(32, 16)]
                sv = xlv + xrv
                ev = jnp.where(sv >= 0.0, sv, 0.2 * sv)
                exv = jnp.exp(_lanesum(ev * attv[0, :]))
                msg[b, pl.ds(0, 16)] = exv * xlv
                msg[b, pl.ds(16, 16)] = exv * auxv
                return _2

            lax.fori_loop(0, B, _edge, None)
            pltpu.sync_copy(msg, num_sp.at[dstv.at[j]], add=True)
            return _c

        lax.fori_loop(0, IBLK, _chunk, None)
        return _

    lax.fori_loop(0, NBLK2, _blk, None)
    plsc.subcore_barrier()
    pltpu.sync_copy(num_sp.at[pl.ds(s * NPT, NPT)],
                    out_hbm.at[c, pl.ds(s * NPT, NPT)])


_stage_b2 = pl.kernel(
    _gat2_body,
    out_type=jax.ShapeDtypeStruct((NC, NPAD, 128), jnp.float32),
    mesh=plsc.VectorSubcoreMesh(core_axis_name="c", subcore_axis_name="s"),
    compiler_params=pltpu.CompilerParams(needs_layout_passes=False),
    scratch_types=[
        pltpu.VMEM((IBLK, B), jnp.int32),
        pltpu.VMEM((IBLK, B), jnp.int32),
        pltpu.VMEM((B, 128), jnp.float32),
        pltpu.VMEM((B, 128), jnp.float32),
        pltpu.VMEM((B, 128), jnp.float32),
        pltpu.VMEM((1, 16), jnp.float32),
        pltpu.VMEM_SHARED((NPAD, 128), jnp.float32),
        pltpu.SemaphoreType.DMA,
        pltpu.SemaphoreType.DMA,
    ],
)


def _head_body(a0_ref, a1_ref, batch_ref, b2_ref, w3_ref, b3_ref, w4_ref,
               b4_ref, o_ref):
    a = a0_ref[...] + a1_ref[...]
    num = a[:, :16]
    den = a[:, 16:17]
    h2 = num / jnp.where(den == 0.0, 1.0, den) + b2_ref[...]
    onehot = (batch_ref[...] == lax.broadcasted_iota(jnp.int32, (1, 64), 1))
    onehot = onehot.astype(jnp.float32)
    sums = lax.dot_general(onehot, h2, (((0,), (0,)), ((), ())),
                           preferred_element_type=jnp.float32)
    cnt = jnp.sum(onehot, axis=0)[:, None]
    pooled = sums / jnp.maximum(cnt, 1.0)
    y = jnp.dot(pooled, w3_ref[...], preferred_element_type=jnp.float32) + b3_ref[...]
    y = jnp.maximum(y, 0.0)
    o_ref[...] = (
        jnp.dot(y, w4_ref[...], preferred_element_type=jnp.float32) + b4_ref[...]
    )


def _stage_d(a0, a1, batch2d, b2, w3, b3, w4, b4):
    return pl.pallas_call(
        _head_body,
        out_shape=jax.ShapeDtypeStruct((64, 1), jnp.float32),
    )(a0, a1, batch2d, b2, w3, b3, w4, b4)


def kernel(x, edge_index, batch, Wl1, bl1, Wr1, br1, att1, bias1,
           Wl2, bl2, Wr2, br2, att2, bias2, W3, b3, W4, b4):
    w1 = jnp.concatenate([Wl1, Wr1], axis=1)
    b1 = jnp.concatenate([bl1, br1])[None, :]
    xlr = _stage_a(x, w1, b1)
    xl_flat = jnp.concatenate([xlr[:, 0:128], xlr[:, 128:256]], axis=0)
    xr_flat = jnp.concatenate([xlr[:, 256:384], xlr[:, 384:512]], axis=0)

    src4a = edge_index[0].reshape(NS, NBLK1, IBLK, B)
    dst4a = edge_index[1].reshape(NS, NBLK1, IBLK, B)
    src4b = edge_index[0].reshape(NC * NS, NBLK2, IBLK, B)
    dst4b = edge_index[1].reshape(NC * NS, NBLK2, IBLK, B)
    zero128 = jnp.zeros((NPAD, 128), jnp.float32)

    num_out, den_out = _stage_b(src4a, dst4a, xl_flat, xr_flat, att1, zero128)
    den8 = den_out.reshape(NC, DND * 16, 8)[:, :N]
    t_tab = _stage_c(num_out[0, :N], num_out[1, :N],
                     den8[0], den8[1], bias1[None, :],
                     Wl2, bl2[None, :], Wr2, br2[None, :])
    aug2 = _stage_b2(src4b, dst4b, t_tab, att2, zero128)
    return _stage_d(aug2[0, :N], aug2[1, :N], batch[:, None], bias2[None, :],
                    W3, b3[None, :], W4, b4[None, :])


# speedup vs baseline: 24.2208x; 2.4102x over previous
import jax
import jax.numpy as jnp
from jax import lax
from jax.experimental import pallas as pl
from jax.experimental.pallas import tpu as pltpu
import jax.experimental.pallas.tpu_sc as plsc

N = 10000
E = 320000
IN = 128
NC = 2
NS = 16
B = 80
IBLK = 5
NBLK1 = E // (NS * IBLK * B)
NBLK2 = E // (NC * NS * IBLK * B)
NPAD = 10112
NPT = NPAD // NS
DND = 640
DPT = DND // NS


def _lane():
    return lax.broadcasted_iota(jnp.int32, (16,), 0)


def _perm(v, k):
    return v.at[_lane() ^ k].get(mode="promise_in_bounds", unique_indices=True)


def _lanesum(v):
    for k in (1, 2, 4, 8):
        v = v + _perm(v, k)
    return v


def _proj_body(x_ref, w_ref, b_ref, o_ref):
    o_ref[...] = (
        jnp.dot(x_ref[...], w_ref[...], preferred_element_type=jnp.float32)
        + b_ref[...]
    )


def _stage_a(x, w, b):
    return pl.pallas_call(
        _proj_body,
        grid=(10,),
        in_specs=[
            pl.BlockSpec((1000, IN), lambda i: (i, 0)),
            pl.BlockSpec((IN, 512), lambda i: (0, 0)),
            pl.BlockSpec((1, 512), lambda i: (0, 0)),
        ],
        out_specs=pl.BlockSpec((1000, 512), lambda i: (i, 0)),
        out_shape=jax.ShapeDtypeStruct((N, 512), jnp.float32),
    )(x, w, b)


def _gat1_body(src_hbm, dst_hbm, xl_hbm, xr_hbm, att_hbm, zero_hbm,
               num_out, den_out,
               srcv, dstv, dgv, dhv, xlg, xrg, msg, attv,
               num_sp, den_sp, sem1, sem2):
    c = lax.axis_index("c")
    s = lax.axis_index("s")
    lane = lax.broadcasted_iota(jnp.int32, (16,), 0)
    pltpu.sync_copy(zero_hbm.at[pl.ds(s * NPT, NPT)],
                    num_sp.at[pl.ds(s * NPT, NPT)])
    pltpu.sync_copy(zero_hbm.at[pl.ds(s * DPT, DPT)],
                    den_sp.at[pl.ds(s * DPT, DPT)])
    pltpu.sync_copy(att_hbm.at[pl.ds(c * 8, 8)], attv)
    plsc.subcore_barrier()
    off = c * N

    def _blk(t, _):
        pltpu.sync_copy(src_hbm.at[s, t], srcv)
        pltpu.sync_copy(dst_hbm.at[s, t], dstv)

        def _adj(i, _a):
            for k in range(B // 16):
                d = dstv[i, pl.ds(k * 16, 16)]
                dgv[i, pl.ds(k * 16, 16)] = d + off
                dhv[i, pl.ds(k * 16, 16)] = d >> 4
                srcv[i, pl.ds(k * 16, 16)] = srcv[i, pl.ds(k * 16, 16)] + off
            return _a

        lax.fori_loop(0, IBLK, _adj, None)

        def _chunk(j, _c):
            cp1 = pltpu.async_copy(xl_hbm.at[srcv.at[j]], xlg, sem1)
            cp2 = pltpu.async_copy(xr_hbm.at[dgv.at[j]], xrg, sem2)
            cp1.wait()
            cp2.wait()

            def _edge(b, _2):
                dsp = plsc.load_gather(
                    dstv,
                    [jnp.broadcast_to(j, (16,)), jnp.broadcast_to(b, (16,))])
                par8 = (dsp & 1) * 8
                grp = (dsp >> 1) & 7
                xls = []
                ts = []
                for h in range(8):
                    xlv = xlg[b, pl.ds(h * 16, 16)]
                    xrv = xrg[b, pl.ds(h * 16, 16)]
                    sv = xlv + xrv
                    ev = jnp.maximum(sv, 0.2 * sv)
                    xls.append(xlv)
                    ts.append(ev * attv[h, :])
                ms = []
                for p in range(4):
                    ua = ts[2 * p] + _perm(ts[2 * p], 1)
                    ub = ts[2 * p + 1] + _perm(ts[2 * p + 1], 1)
                    ms.append(jnp.where((lane & 1) == 0, ua, ub))
                ws = [m + _perm(m, 2) for m in ms]
                vs = [jnp.where((lane & 2) == 0, ws[0], ws[1]),
                      jnp.where((lane & 2) == 0, ws[2], ws[3])]
                zs = [v + _perm(v, 4) for v in vs]
                z = jnp.where((lane & 4) == 0, zs[0], zs[1])
                full = jnp.exp(z + _perm(z, 8))
                for h in range(8):
                    exh = full.at[jnp.full((16,), h, jnp.int32)].get(
                        mode="promise_in_bounds")
                    msg[b, pl.ds(h * 16, 16)] = exh * xls[h]
                placed = jnp.where((lane >= 8) == (par8 == 8), full, 0.0)
                for g in range(8):
                    xrg[b, pl.ds(g * 16, 16)] = jnp.where(grp == g, placed, 0.0)
                return _2

            lax.fori_loop(0, B, _edge, None)
            pltpu.sync_copy(msg, num_sp.at[dstv.at[j]], add=True)
            pltpu.sync_copy(xrg, den_sp.at[dhv.at[j]], add=True)
            return _c

        lax.fori_loop(0, IBLK, _chunk, None)
        return _

    lax.fori_loop(0, NBLK1, _blk, None)
    plsc.subcore_barrier()
    pltpu.sync_copy(num_sp.at[pl.ds(s * NPT, NPT)],
                    num_out.at[c, pl.ds(s * NPT, NPT)])
    pltpu.sync_copy(den_sp.at[pl.ds(s * DPT, DPT)],
                    den_out.at[c, pl.ds(s * DPT, DPT)])


_stage_b = pl.kernel(
    _gat1_body,
    out_type=[
        jax.ShapeDtypeStruct((NC, NPAD, 128), jnp.float32),
        jax.ShapeDtypeStruct((NC, DND, 128), jnp.float32),
    ],
    mesh=plsc.VectorSubcoreMesh(core_axis_name="c", subcore_axis_name="s"),
    compiler_params=pltpu.CompilerParams(needs_layout_passes=False),
    scratch_types=[
        pltpu.VMEM((IBLK, B), jnp.int32),
        pltpu.VMEM((IBLK, B), jnp.int32),
        pltpu.VMEM((IBLK, B), jnp.int32),
        pltpu.VMEM((IBLK, B), jnp.int32),
        pltpu.VMEM((B, 128), jnp.float32),
        pltpu.VMEM((B, 128), jnp.float32),
        pltpu.VMEM((B, 128), jnp.float32),
        pltpu.VMEM((8, 16), jnp.float32),
        pltpu.VMEM_SHARED((NPAD, 128), jnp.float32),
        pltpu.VMEM_SHARED((DND, 128), jnp.float32),
        pltpu.SemaphoreType.DMA,
        pltpu.SemaphoreType.DMA,
    ],
)


def _mid_body(a0_ref, a1_ref, d0_ref, d1_ref, b1_ref, wl_ref, bl_ref,
              wr_ref, br_ref, t_ref):
    num = jnp.concatenate([a0_ref[...], a1_ref[...]], axis=1)
    den16 = jnp.concatenate([d0_ref[...], d1_ref[...]], axis=1)
    rows = num.shape[0]
    den = jnp.reshape(
        jnp.broadcast_to(den16[:, :, None], (rows, 16, 16)), (rows, 256)
    )
    h = num / jnp.where(den == 0.0, 1.0, den) + b1_ref[...]
    h = jnp.where(h > 0.0, h, jnp.exp(h) - 1.0)
    xl2 = jnp.dot(h, wl_ref[...], preferred_element_type=jnp.float32) + bl_ref[...]
    xr2 = jnp.dot(h, wr_ref[...], preferred_element_type=jnp.float32) + br_ref[...]
    t_ref[...] = jnp.concatenate(
        [xl2,
         jnp.ones((rows, 1), jnp.float32), jnp.zeros((rows, 15), jnp.float32),
         xr2,
         jnp.zeros((rows, 80), jnp.float32)],
        axis=1,
    )


def _stage_c(a0, a1, d0, d1, b1, wl, bl, wr, br):
    return pl.pallas_call(
        _mid_body,
        grid=(10,),
        in_specs=[
            pl.BlockSpec((1000, 128), lambda i: (i, 0)),
            pl.BlockSpec((1000, 128), lambda i: (i, 0)),
            pl.BlockSpec((1000, 8), lambda i: (i, 0)),
            pl.BlockSpec((1000, 8), lambda i: (i, 0)),
            pl.BlockSpec((1, 256), lambda i: (0, 0)),
            pl.BlockSpec((256, 16), lambda i: (0, 0)),
            pl.BlockSpec((1, 16), lambda i: (0, 0)),
            pl.BlockSpec((256, 16), lambda i: (0, 0)),
            pl.BlockSpec((1, 16), lambda i: (0, 0)),
        ],
        out_specs=pl.BlockSpec((1000, 128), lambda i: (i, 0)),
        out_shape=jax.ShapeDtypeStruct((N, 128), jnp.float32),
    )(a0, a1, d0, d1, b1, wl, bl, wr, br)


def _gat2_body(src_hbm, dst_hbm, t_hbm, att_hbm, zero_hbm, out_hbm,
               srcv, dstv, xlg, xrg, msg, attv, num_sp, sem1, sem2):
    c = lax.axis_index("c")
    s = lax.axis_index("s")
    pltpu.sync_copy(zero_hbm.at[pl.ds(s * NPT, NPT)],
                    num_sp.at[pl.ds(s * NPT, NPT)])
    pltpu.sync_copy(att_hbm, attv)

    def _mzero(b, _):
        for k in range(2, 8):
            msg[b, pl.ds(k * 16, 16)] = jnp.zeros((16,), jnp.float32)
        return _

    lax.fori_loop(0, B, _mzero, None)
    plsc.subcore_barrier()
    w = c * NS + s

    def _blk(t, _):
        pltpu.sync_copy(src_hbm.at[w, t], srcv)
        pltpu.sync_copy(dst_hbm.at[w, t], dstv)

        def _chunk(j, _c):
            cp1 = pltpu.async_copy(t_hbm.at[srcv.at[j]], xlg, sem1)
            cp2 = pltpu.async_copy(t_hbm.at[dstv.at[j]], xrg, sem2)
            cp1.wait()
            cp2.wait()

            def _edge(b, _2):
                xlv = xlg[b, pl.ds(0, 16)]
                auxv = xlg[b, pl.ds(16, 16)]
                xrv = xrg[b, pl.ds(32, 16)]
                sv = xlv + xrv
                ev = jnp.maximum(sv, 0.2 * sv)
                exv = jnp.exp(_lanesum(ev * attv[0, :]))
                msg[b, pl.ds(0, 16)] = exv * xlv
                msg[b, pl.ds(16, 16)] = exv * auxv
                return _2

            lax.fori_loop(0, B, _edge, None)
            pltpu.sync_copy(msg, num_sp.at[dstv.at[j]], add=True)
            return _c

        lax.fori_loop(0, IBLK, _chunk, None)
        return _

    lax.fori_loop(0, NBLK2, _blk, None)
    plsc.subcore_barrier()
    pltpu.sync_copy(num_sp.at[pl.ds(s * NPT, NPT)],
                    out_hbm.at[c, pl.ds(s * NPT, NPT)])


_stage_b2 = pl.kernel(
    _gat2_body,
    out_type=jax.ShapeDtypeStruct((NC, NPAD, 128), jnp.float32),
    mesh=plsc.VectorSubcoreMesh(core_axis_name="c", subcore_axis_name="s"),
    compiler_params=pltpu.CompilerParams(needs_layout_passes=False),
    scratch_types=[
        pltpu.VMEM((IBLK, B), jnp.int32),
        pltpu.VMEM((IBLK, B), jnp.int32),
        pltpu.VMEM((B, 128), jnp.float32),
        pltpu.VMEM((B, 128), jnp.float32),
        pltpu.VMEM((B, 128), jnp.float32),
        pltpu.VMEM((1, 16), jnp.float32),
        pltpu.VMEM_SHARED((NPAD, 128), jnp.float32),
        pltpu.SemaphoreType.DMA,
        pltpu.SemaphoreType.DMA,
    ],
)


def _head_body(a0_ref, a1_ref, batch_ref, b2_ref, w3_ref, b3_ref, w4_ref,
               b4_ref, o_ref):
    a = a0_ref[...] + a1_ref[...]
    num = a[:, :16]
    den = a[:, 16:17]
    h2 = num / jnp.where(den == 0.0, 1.0, den) + b2_ref[...]
    onehot = (batch_ref[...] == lax.broadcasted_iota(jnp.int32, (1, 64), 1))
    onehot = onehot.astype(jnp.float32)
    sums = lax.dot_general(onehot, h2, (((0,), (0,)), ((), ())),
                           preferred_element_type=jnp.float32)
    cnt = jnp.sum(onehot, axis=0)[:, None]
    pooled = sums / jnp.maximum(cnt, 1.0)
    y = jnp.dot(pooled, w3_ref[...], preferred_element_type=jnp.float32) + b3_ref[...]
    y = jnp.maximum(y, 0.0)
    o_ref[...] = (
        jnp.dot(y, w4_ref[...], preferred_element_type=jnp.float32) + b4_ref[...]
    )


def _stage_d(a0, a1, batch2d, b2, w3, b3, w4, b4):
    return pl.pallas_call(
        _head_body,
        out_shape=jax.ShapeDtypeStruct((64, 1), jnp.float32),
    )(a0, a1, batch2d, b2, w3, b3, w4, b4)


def kernel(x, edge_index, batch, Wl1, bl1, Wr1, br1, att1, bias1,
           Wl2, bl2, Wr2, br2, att2, bias2, W3, b3, W4, b4):
    w1 = jnp.concatenate([Wl1, Wr1], axis=1)
    b1 = jnp.concatenate([bl1, br1])[None, :]
    xlr = _stage_a(x, w1, b1)
    xl_flat = jnp.concatenate([xlr[:, 0:128], xlr[:, 128:256]], axis=0)
    xr_flat = jnp.concatenate([xlr[:, 256:384], xlr[:, 384:512]], axis=0)

    src4a = edge_index[0].reshape(NS, NBLK1, IBLK, B)
    dst4a = edge_index[1].reshape(NS, NBLK1, IBLK, B)
    src4b = edge_index[0].reshape(NC * NS, NBLK2, IBLK, B)
    dst4b = edge_index[1].reshape(NC * NS, NBLK2, IBLK, B)
    zero128 = jnp.zeros((NPAD, 128), jnp.float32)

    num_out, den_out = _stage_b(src4a, dst4a, xl_flat, xr_flat, att1, zero128)
    den8 = den_out.reshape(NC, DND * 16, 8)[:, :N]
    t_tab = _stage_c(num_out[0, :N], num_out[1, :N],
                     den8[0], den8[1], bias1[None, :],
                     Wl2, bl2[None, :], Wr2, br2[None, :])
    aug2 = _stage_b2(src4b, dst4b, t_tab, att2, zero128)
    return _stage_d(aug2[0, :N], aug2[1, :N], batch[:, None], bias2[None, :],
                    W3, b3[None, :], W4, b4[None, :])


# speedup vs baseline: 26.5446x; 1.0959x over previous
import jax
import jax.numpy as jnp
from jax import lax
from jax.experimental import pallas as pl
from jax.experimental.pallas import tpu as pltpu
import jax.experimental.pallas.tpu_sc as plsc

N = 10000
E = 320000
IN = 128
NC = 2
NS = 16
B = 80
IBLK = 5
NBLK1 = E // (NS * IBLK * B)
NBLK2 = E // (NC * NS * IBLK * B)
NPAD = 10112
NPT = NPAD // NS
DND = 640
DPT = DND // NS


def _lane():
    return lax.broadcasted_iota(jnp.int32, (16,), 0)


def _perm(v, k):
    return v.at[_lane() ^ k].get(mode="promise_in_bounds", unique_indices=True)


def _lanesum(v):
    for k in (1, 2, 4, 8):
        v = v + _perm(v, k)
    return v


def _proj_body(x_ref, w_ref, b_ref, o_ref):
    o_ref[...] = (
        jnp.dot(x_ref[...], w_ref[...], preferred_element_type=jnp.float32)
        + b_ref[...]
    )


def _stage_a(x, w, b):
    return pl.pallas_call(
        _proj_body,
        grid=(10,),
        in_specs=[
            pl.BlockSpec((1000, IN), lambda i: (i, 0)),
            pl.BlockSpec((IN, 512), lambda i: (0, 0)),
            pl.BlockSpec((1, 512), lambda i: (0, 0)),
        ],
        out_specs=pl.BlockSpec((1000, 512), lambda i: (i, 0)),
        out_shape=jax.ShapeDtypeStruct((N, 512), jnp.float32),
    )(x, w, b)


def _gat1_body(src_hbm, dst_hbm, xl_hbm, xr_hbm, att_hbm, zero_hbm,
               num_out, den_out,
               srcv, dstv, dgv, dhv, xlg, xrg, msg, attv,
               num_sp, den_sp, sem1, sem2):
    c = lax.axis_index("c")
    s = lax.axis_index("s")
    lane = lax.broadcasted_iota(jnp.int32, (16,), 0)
    pltpu.sync_copy(zero_hbm.at[pl.ds(s * NPT, NPT)],
                    num_sp.at[pl.ds(s * NPT, NPT)])
    pltpu.sync_copy(zero_hbm.at[pl.ds(s * DPT, DPT)],
                    den_sp.at[pl.ds(s * DPT, DPT)])
    pltpu.sync_copy(att_hbm.at[pl.ds(c * 8, 8)], attv)
    plsc.subcore_barrier()
    off = c * N

    def _blk(t, _):
        pltpu.sync_copy(src_hbm.at[s, t], srcv)
        pltpu.sync_copy(dst_hbm.at[s, t], dstv)

        def _adj(i, _a):
            for k in range(B // 16):
                d = dstv[i, pl.ds(k * 16, 16)]
                dgv[i, pl.ds(k * 16, 16)] = d + off
                dhv[i, pl.ds(k * 16, 16)] = d >> 4
                srcv[i, pl.ds(k * 16, 16)] = srcv[i, pl.ds(k * 16, 16)] + off
            return _a

        lax.fori_loop(0, IBLK, _adj, None)

        def _chunk(j, _c):
            cp1 = pltpu.async_copy(xl_hbm.at[srcv.at[j]], xlg, sem1)
            cp2 = pltpu.async_copy(xr_hbm.at[dgv.at[j]], xrg, sem2)
            cp1.wait()
            cp2.wait()

            def _edge(b, _2):
                dsp = plsc.load_gather(
                    dstv,
                    [jnp.broadcast_to(j, (16,)), jnp.broadcast_to(b, (16,))])
                par8 = (dsp & 1) * 8
                grp = (dsp >> 1) & 7
                xls = []
                ts = []
                for h in range(8):
                    xlv = xlg[b, pl.ds(h * 16, 16)]
                    xrv = xrg[b, pl.ds(h * 16, 16)]
                    sv = xlv + xrv
                    ev = jnp.maximum(sv, 0.2 * sv)
                    xls.append(xlv)
                    ts.append(ev * attv[h, :])
                ms = []
                for p in range(4):
                    ua = ts[2 * p] + _perm(ts[2 * p], 1)
                    ub = ts[2 * p + 1] + _perm(ts[2 * p + 1], 1)
                    ms.append(jnp.where((lane & 1) == 0, ua, ub))
                ws = [m + _perm(m, 2) for m in ms]
                vs = [jnp.where((lane & 2) == 0, ws[0], ws[1]),
                      jnp.where((lane & 2) == 0, ws[2], ws[3])]
                zs = [v + _perm(v, 4) for v in vs]
                z = jnp.where((lane & 4) == 0, zs[0], zs[1])
                full = jnp.exp(z + _perm(z, 8))
                for h in range(8):
                    exh = full.at[jnp.full((16,), h, jnp.int32)].get(
                        mode="promise_in_bounds")
                    msg[b, pl.ds(h * 16, 16)] = exh * xls[h]
                placed = jnp.where((lane >= 8) == (par8 == 8), full, 0.0)
                for g in range(8):
                    xrg[b, pl.ds(g * 16, 16)] = jnp.where(grp == g, placed, 0.0)
                return _2

            lax.fori_loop(0, B, _edge, None)
            cp3 = pltpu.async_copy(msg, num_sp.at[dstv.at[j]], sem1, add=True)
            cp4 = pltpu.async_copy(xrg, den_sp.at[dhv.at[j]], sem2, add=True)
            cp3.wait()
            cp4.wait()
            return _c

        lax.fori_loop(0, IBLK, _chunk, None)
        return _

    lax.fori_loop(0, NBLK1, _blk, None)
    plsc.subcore_barrier()
    pltpu.sync_copy(num_sp.at[pl.ds(s * NPT, NPT)],
                    num_out.at[c, pl.ds(s * NPT, NPT)])
    pltpu.sync_copy(den_sp.at[pl.ds(s * DPT, DPT)],
                    den_out.at[c, pl.ds(s * DPT, DPT)])


_stage_b = pl.kernel(
    _gat1_body,
    out_type=[
        jax.ShapeDtypeStruct((NC, NPAD, 128), jnp.float32),
        jax.ShapeDtypeStruct((NC, DND, 128), jnp.float32),
    ],
    mesh=plsc.VectorSubcoreMesh(core_axis_name="c", subcore_axis_name="s"),
    compiler_params=pltpu.CompilerParams(needs_layout_passes=False),
    scratch_types=[
        pltpu.VMEM((IBLK, B), jnp.int32),
        pltpu.VMEM((IBLK, B), jnp.int32),
        pltpu.VMEM((IBLK, B), jnp.int32),
        pltpu.VMEM((IBLK, B), jnp.int32),
        pltpu.VMEM((B, 128), jnp.float32),
        pltpu.VMEM((B, 128), jnp.float32),
        pltpu.VMEM((B, 128), jnp.float32),
        pltpu.VMEM((8, 16), jnp.float32),
        pltpu.VMEM_SHARED((NPAD, 128), jnp.float32),
        pltpu.VMEM_SHARED((DND, 128), jnp.float32),
        pltpu.SemaphoreType.DMA,
        pltpu.SemaphoreType.DMA,
    ],
)


def _mid_body(a0_ref, a1_ref, d0_ref, d1_ref, b1_ref, wl_ref, bl_ref,
              wr_ref, br_ref, t_ref):
    num = jnp.concatenate([a0_ref[...], a1_ref[...]], axis=1)
    den16 = jnp.concatenate([d0_ref[...], d1_ref[...]], axis=1)
    rows = num.shape[0]
    den = jnp.reshape(
        jnp.broadcast_to(den16[:, :, None], (rows, 16, 16)), (rows, 256)
    )
    h = num / jnp.where(den == 0.0, 1.0, den) + b1_ref[...]
    h = jnp.where(h > 0.0, h, jnp.exp(h) - 1.0)
    xl2 = jnp.dot(h, wl_ref[...], preferred_element_type=jnp.float32) + bl_ref[...]
    xr2 = jnp.dot(h, wr_ref[...], preferred_element_type=jnp.float32) + br_ref[...]
    t_ref[...] = jnp.concatenate(
        [xl2,
         jnp.ones((rows, 1), jnp.float32), jnp.zeros((rows, 15), jnp.float32),
         xr2,
         jnp.zeros((rows, 80), jnp.float32)],
        axis=1,
    )


def _stage_c(a0, a1, d0, d1, b1, wl, bl, wr, br):
    return pl.pallas_call(
        _mid_body,
        grid=(10,),
        in_specs=[
            pl.BlockSpec((1000, 128), lambda i: (i, 0)),
            pl.BlockSpec((1000, 128), lambda i: (i, 0)),
            pl.BlockSpec((1000, 8), lambda i: (i, 0)),
            pl.BlockSpec((1000, 8), lambda i: (i, 0)),
            pl.BlockSpec((1, 256), lambda i: (0, 0)),
            pl.BlockSpec((256, 16), lambda i: (0, 0)),
            pl.BlockSpec((1, 16), lambda i: (0, 0)),
            pl.BlockSpec((256, 16), lambda i: (0, 0)),
            pl.BlockSpec((1, 16), lambda i: (0, 0)),
        ],
        out_specs=pl.BlockSpec((1000, 128), lambda i: (i, 0)),
        out_shape=jax.ShapeDtypeStruct((N, 128), jnp.float32),
    )(a0, a1, d0, d1, b1, wl, bl, wr, br)


def _gat2_body(src_hbm, dst_hbm, t_hbm, att_hbm, zero_hbm, out_hbm,
               srcv, dstv, xlg, xrg, msg, attv, num_sp, sem1, sem2):
    c = lax.axis_index("c")
    s = lax.axis_index("s")
    pltpu.sync_copy(zero_hbm.at[pl.ds(s * NPT, NPT)],
                    num_sp.at[pl.ds(s * NPT, NPT)])
    pltpu.sync_copy(att_hbm, attv)

    def _mzero(b, _):
        for k in range(2, 8):
            msg[b, pl.ds(k * 16, 16)] = jnp.zeros((16,), jnp.float32)
        return _

    lax.fori_loop(0, B, _mzero, None)
    plsc.subcore_barrier()
    w = c * NS + s

    def _blk(t, _):
        pltpu.sync_copy(src_hbm.at[w, t], srcv)
        pltpu.sync_copy(dst_hbm.at[w, t], dstv)

        def _chunk(j, _c):
            cp1 = pltpu.async_copy(t_hbm.at[srcv.at[j]], xlg, sem1)
            cp2 = pltpu.async_copy(t_hbm.at[dstv.at[j]], xrg, sem2)
            cp1.wait()
            cp2.wait()

            def _e8(g, _2):
                lane = lax.broadcasted_iota(jnp.int32, (16,), 0)
                xls = []
                auxs = []
                ts = []
                for e in range(8):
                    b = g * 8 + e
                    xlv = xlg[b, pl.ds(0, 16)]
                    auxv = xlg[b, pl.ds(16, 16)]
                    xrv = xrg[b, pl.ds(32, 16)]
                    sv = xlv + xrv
                    ev = jnp.maximum(sv, 0.2 * sv)
                    xls.append(xlv)
                    auxs.append(auxv)
                    ts.append(ev * attv[0, :])
                ms = []
                for p in range(4):
                    ua = ts[2 * p] + _perm(ts[2 * p], 1)
                    ub = ts[2 * p + 1] + _perm(ts[2 * p + 1], 1)
                    ms.append(jnp.where((lane & 1) == 0, ua, ub))
                ws = [m + _perm(m, 2) for m in ms]
                vs = [jnp.where((lane & 2) == 0, ws[0], ws[1]),
                      jnp.where((lane & 2) == 0, ws[2], ws[3])]
                zs = [v + _perm(v, 4) for v in vs]
                z = jnp.where((lane & 4) == 0, zs[0], zs[1])
                exf = jnp.exp(z + _perm(z, 8))
                for e in range(8):
                    b = g * 8 + e
                    exh = exf.at[jnp.full((16,), e, jnp.int32)].get(
                        mode="promise_in_bounds")
                    msg[b, pl.ds(0, 16)] = exh * xls[e]
                    msg[b, pl.ds(16, 16)] = exh * auxs[e]
                return _2

            lax.fori_loop(0, B // 8, _e8, None)
            pltpu.sync_copy(msg, num_sp.at[dstv.at[j]], add=True)
            return _c

        lax.fori_loop(0, IBLK, _chunk, None)
        return _

    lax.fori_loop(0, NBLK2, _blk, None)
    plsc.subcore_barrier()
    pltpu.sync_copy(num_sp.at[pl.ds(s * NPT, NPT)],
                    out_hbm.at[c, pl.ds(s * NPT, NPT)])


_stage_b2 = pl.kernel(
    _gat2_body,
    out_type=jax.ShapeDtypeStruct((NC, NPAD, 128), jnp.float32),
    mesh=plsc.VectorSubcoreMesh(core_axis_name="c", subcore_axis_name="s"),
    compiler_params=pltpu.CompilerParams(needs_layout_passes=False),
    scratch_types=[
        pltpu.VMEM((IBLK, B), jnp.int32),
        pltpu.VMEM((IBLK, B), jnp.int32),
        pltpu.VMEM((B, 128), jnp.float32),
        pltpu.VMEM((B, 128), jnp.float32),
        pltpu.VMEM((B, 128), jnp.float32),
        pltpu.VMEM((1, 16), jnp.float32),
        pltpu.VMEM_SHARED((NPAD, 128), jnp.float32),
        pltpu.SemaphoreType.DMA,
        pltpu.SemaphoreType.DMA,
    ],
)


def _head_body(a0_ref, a1_ref, batch_ref, b2_ref, w3_ref, b3_ref, w4_ref,
               b4_ref, o_ref):
    a = a0_ref[...] + a1_ref[...]
    num = a[:, :16]
    den = a[:, 16:17]
    h2 = num / jnp.where(den == 0.0, 1.0, den) + b2_ref[...]
    onehot = (batch_ref[...] == lax.broadcasted_iota(jnp.int32, (1, 64), 1))
    onehot = onehot.astype(jnp.float32)
    sums = lax.dot_general(onehot, h2, (((0,), (0,)), ((), ())),
                           preferred_element_type=jnp.float32)
    cnt = jnp.sum(onehot, axis=0)[:, None]
    pooled = sums / jnp.maximum(cnt, 1.0)
    y = jnp.dot(pooled, w3_ref[...], preferred_element_type=jnp.float32) + b3_ref[...]
    y = jnp.maximum(y, 0.0)
    o_ref[...] = (
        jnp.dot(y, w4_ref[...], preferred_element_type=jnp.float32) + b4_ref[...]
    )


def _stage_d(a0, a1, batch2d, b2, w3, b3, w4, b4):
    return pl.pallas_call(
        _head_body,
        out_shape=jax.ShapeDtypeStruct((64, 1), jnp.float32),
    )(a0, a1, batch2d, b2, w3, b3, w4, b4)


def kernel(x, edge_index, batch, Wl1, bl1, Wr1, br1, att1, bias1,
           Wl2, bl2, Wr2, br2, att2, bias2, W3, b3, W4, b4):
    w1 = jnp.concatenate([Wl1, Wr1], axis=1)
    b1 = jnp.concatenate([bl1, br1])[None, :]
    xlr = _stage_a(x, w1, b1)
    xl_flat = jnp.concatenate([xlr[:, 0:128], xlr[:, 128:256]], axis=0)
    xr_flat = jnp.concatenate([xlr[:, 256:384], xlr[:, 384:512]], axis=0)

    src4a = edge_index[0].reshape(NS, NBLK1, IBLK, B)
    dst4a = edge_index[1].reshape(NS, NBLK1, IBLK, B)
    src4b = edge_index[0].reshape(NC * NS, NBLK2, IBLK, B)
    dst4b = edge_index[1].reshape(NC * NS, NBLK2, IBLK, B)
    zero128 = jnp.zeros((NPAD, 128), jnp.float32)

    num_out, den_out = _stage_b(src4a, dst4a, xl_flat, xr_flat, att1, zero128)
    den8 = den_out.reshape(NC, DND * 16, 8)[:, :N]
    t_tab = _stage_c(num_out[0, :N], num_out[1, :N],
                     den8[0], den8[1], bias1[None, :],
                     Wl2, bl2[None, :], Wr2, br2[None, :])
    aug2 = _stage_b2(src4b, dst4b, t_tab, att2, zero128)
    return _stage_d(aug2[0, :N], aug2[1, :N], batch[:, None], bias2[None, :],
                    W3, b3[None, :], W4, b4[None, :])
